# baseline (speedup 1.0000x reference)
import functools
import math

import jax
import jax.numpy as jnp
from jax import lax
from jax.experimental import pallas as pl
from jax.experimental.pallas import tpu as pltpu
from jax.experimental.pallas import tpu_sc as plsc

EMB = 64
SCALE = math.sqrt(EMB)
CHUNK = 128


def kernel(tokens, table):
    B, L = tokens.shape
    vocab, emb = table.shape
    assert emb == EMB
    n = B * L
    info = plsc.get_sparse_core_info()
    num_workers = info.num_cores * info.num_subcores
    assert n % (num_workers * CHUNK) == 0
    n_per_w = n // num_workers
    steps = n_per_w // CHUNK

    idx = tokens.reshape(n).astype(jnp.int32)

    mesh = plsc.VectorSubcoreMesh(core_axis_name="c", subcore_axis_name="s")

    @functools.partial(
        pl.kernel,
        out_type=jax.ShapeDtypeStruct((n, EMB), jnp.float32),
        mesh=mesh,
        scratch_types=[
            pltpu.VMEM((CHUNK,), jnp.int32),
            pltpu.VMEM((CHUNK, EMB), jnp.float32),
            pltpu.SemaphoreType.DMA,
        ],
        compiler_params=pltpu.CompilerParams(use_tc_tiling_on_sc=False),
    )
    def emb_lookup(idx_hbm, table_hbm, out_hbm, idx_v, rows_v, sem):
        wid = lax.axis_index("s") * info.num_cores + lax.axis_index("c")
        base = wid * n_per_w

        def step(c, carry):
            off = base + c * CHUNK
            pltpu.sync_copy(idx_hbm.at[pl.ds(off, CHUNK)], idx_v)
            pltpu.async_copy(table_hbm.at[idx_v], rows_v, sem).wait()

            def scale_row(r, carry2):
                for k in range(EMB // 16):
                    sl = pl.ds(k * 16, 16)
                    rows_v[r, sl] = rows_v[r, sl] * SCALE
                return carry2

            lax.fori_loop(0, CHUNK, scale_row, 0)
            pltpu.sync_copy(rows_v, out_hbm.at[pl.ds(off, CHUNK)])
            return carry

        lax.fori_loop(0, steps, step, 0)

    out = emb_lookup(idx, table)
    return out.reshape(B, L, EMB)


# speedup vs baseline: 1.2488x; 1.2488x over previous
import functools
import math

import jax
import jax.numpy as jnp
from jax import lax
from jax.experimental import pallas as pl
from jax.experimental.pallas import tpu as pltpu
from jax.experimental.pallas import tpu_sc as plsc

EMB = 64
SCALE = math.sqrt(EMB)
CHUNK = 128
NBUF = 4
LEAD = 2


def kernel(tokens, table):
    B, L = tokens.shape
    vocab, emb = table.shape
    assert emb == EMB
    n = B * L
    info = plsc.get_sparse_core_info()
    num_workers = info.num_cores * info.num_subcores
    assert n % (num_workers * CHUNK) == 0
    n_per_w = n // num_workers
    steps = n_per_w // CHUNK
    assert steps % NBUF == 0 and steps >= 2 * NBUF

    idx = tokens.reshape(num_workers, steps, CHUNK).astype(jnp.int32)

    mesh = plsc.VectorSubcoreMesh(core_axis_name="c", subcore_axis_name="s")

    @functools.partial(
        pl.kernel,
        out_type=jax.ShapeDtypeStruct((n, EMB), jnp.float32),
        mesh=mesh,
        scratch_types=[
            pltpu.VMEM((steps, CHUNK), jnp.int32),
            [pltpu.VMEM((CHUNK, EMB), jnp.float32) for _ in range(NBUF)],
            [pltpu.SemaphoreType.DMA for _ in range(NBUF)],
            [pltpu.SemaphoreType.DMA for _ in range(NBUF)],
        ],
        compiler_params=pltpu.CompilerParams(use_tc_tiling_on_sc=False),
    )
    def emb_lookup(idx_hbm, table_hbm, out_hbm, idx_all, rows, gsem, wsem):
        wid = lax.axis_index("s") * info.num_cores + lax.axis_index("c")
        base = wid * n_per_w

        pltpu.sync_copy(idx_hbm.at[wid], idx_all)

        def fire_gather(c, b):
            pltpu.async_copy(table_hbm.at[idx_all.at[c]], rows[b], gsem[b])

        def wait_gather(c, b):
            pltpu.make_async_copy(
                table_hbm.at[idx_all.at[c]], rows[b], gsem[b]
            ).wait()

        def out_slice(c):
            return out_hbm.at[pl.ds(base + c * CHUNK, CHUNK)]

        def fire_wb(c, b):
            pltpu.async_copy(rows[b], out_slice(c), wsem[b])

        def wait_wb(c, b):
            pltpu.make_async_copy(rows[b], out_slice(c), wsem[b]).wait()

        def scale(rref):
            def row(r, carry):
                for k in range(EMB // 16):
                    sl = pl.ds(k * 16, 16)
                    rref[r, sl] = rref[r, sl] * SCALE
                return carry

            lax.fori_loop(0, CHUNK, row, 0, unroll=4)

        def process(c, b):
            wait_gather(c, b)
            scale(rows[b])
            fire_wb(c, b)

        for c in range(LEAD):
            fire_gather(c, c % NBUF)
        for c in range(LEAD):
            process(c, c % NBUF)
            fire_gather(c + LEAD, (c + LEAD) % NBUF)

        def outer(i, carry):
            c0 = LEAD + i * NBUF
            for j in range(NBUF):
                c = c0 + j
                b = (LEAD + j) % NBUF
                process(c, b)
                br = j % NBUF
                wait_wb(c - LEAD, br)
                fire_gather(c + LEAD, br)
            return carry

        lax.fori_loop(0, (steps - 2 * LEAD) // NBUF, outer, 0)

        for c in range(steps - LEAD, steps):
            process(c, c % NBUF)
        for c in range(steps - NBUF, steps):
            wait_wb(c, c % NBUF)

    out = emb_lookup(idx, table)
    return out.reshape(B, L, EMB)
